# baseline (speedup 1.0000x reference)
import functools

import jax
import jax.numpy as jnp
from jax import lax
from jax.experimental import pallas as pl
from jax.experimental.pallas import tpu as pltpu
from jax.experimental.pallas import tpu_sc as plsc

_R1, _RATIO1, _K1 = 0.5, 0.5, 32
_R2, _RATIO2, _K2 = 1.0, 0.25, 32

_NC, _NS, _NL = 2, 16, 16
_NW = _NC * _NS


def _split_body(x_ref, m_ref, feat_ref, pos_ref):
    x = x_ref[...]
    sel = m_ref[...] > 0
    feat_ref[...] = jnp.where(sel, x[:, 3:], 0.0)
    pos_ref[...] = jnp.where(sel, x[:, :3], 0.0)


def _split(x, mask):
    b, n, f = x.shape
    x2 = x.reshape(b * n, f)
    m2 = mask.reshape(b * n, 1).astype(jnp.int32)
    feat, pos = pl.pallas_call(
        _split_body,
        out_shape=(
            jax.ShapeDtypeStruct((b * n, f - 3), x.dtype),
            jax.ShapeDtypeStruct((b * n, 3), x.dtype),
        ),
    )(x2, m2)
    return feat, pos


def _fps_body(px_ref, py_ref, pz_ref, out_ref, *, s):
    px = px_ref[...]
    py = py_ref[...]
    pz = pz_ref[...]
    b, n = px.shape
    lane = lax.broadcasted_iota(jnp.int32, (b, n), 1)
    lane_s = lax.broadcasted_iota(jnp.int32, (b, s), 1)

    def step(t, carry):
        d, lx, ly, lz, acc = carry
        dx = px - lx
        dy = py - ly
        dz = pz - lz
        sq = (dx * dx + dy * dy) + dz * dz
        nd = jnp.minimum(d, sq)
        m = jnp.max(nd, axis=1, keepdims=True)
        cand = jnp.where(nd == m, lane, n)
        nxt = jnp.min(cand, axis=1, keepdims=True)
        acc = jnp.where(lane_s == t, nxt, acc)
        oh = lane == nxt
        nlx = jnp.sum(jnp.where(oh, px, 0.0), axis=1, keepdims=True)
        nly = jnp.sum(jnp.where(oh, py, 0.0), axis=1, keepdims=True)
        nlz = jnp.sum(jnp.where(oh, pz, 0.0), axis=1, keepdims=True)
        return nd, nlx, nly, nlz, acc

    def step2(t2, carry):
        carry = step(t2 * 2 + 1, carry)
        return step(t2 * 2 + 2, carry)

    init = (
        jnp.full((b, n), jnp.inf, dtype=jnp.float32),
        px[:, 0:1], py[:, 0:1], pz[:, 0:1],
        jnp.zeros((b, s), dtype=jnp.int32),
    )
    carry = lax.fori_loop(0, (s - 2) // 2, step2, init)
    carry = step(s - 1, carry)
    out_ref[...] = carry[4]


def _fps_pallas(px, py, pz, s):
    b, n = px.shape
    return pl.pallas_call(
        functools.partial(_fps_body, s=s),
        out_shape=jax.ShapeDtypeStruct((b, s), jnp.int32),
    )(px, py, pz)


def _sc_prep_body(px_h, py_h, pz_h, feat_h, idx_h,
                  mlp_h, valid_h, cx_h, cy_h, cz_h,
                  px_v, py_v, pz_v, feat_v, idx_v,
                  cx_v, cy_v, cz_v, msk_v, nb_v, stage_v, stvld_v,
                  *, b, n, s, k, cf, r2, slot_major, feat_cm, dpad):
    D = cf + 3
    C = b * s
    chunk = C // _NW
    wpb = _NW // b
    ngrp = chunk // _NL

    cidx = lax.axis_index("c")
    sidx = lax.axis_index("s")
    wid = sidx * _NC + cidx
    bid = wid // wpb
    base_c = wid * chunk

    pltpu.sync_copy(px_h.at[bid], px_v)
    pltpu.sync_copy(py_h.at[bid], py_v)
    pltpu.sync_copy(pz_h.at[bid], pz_v)
    if feat_cm:
        pltpu.sync_copy(feat_h.at[:, pl.ds(bid * n, n)], feat_v)
    else:
        pltpu.sync_copy(feat_h.at[bid], feat_v)
    pltpu.sync_copy(idx_h.at[pl.ds(base_c, chunk)], idx_v)

    iota = lax.iota(jnp.int32, _NL)
    zeros16 = jnp.zeros((_NL,), jnp.float32)

    def cgrp(g, _):
        iv = idx_v[pl.ds(g * _NL, _NL)]
        cx_v[pl.ds(g * _NL, _NL)] = plsc.load_gather(px_v, [iv])
        cy_v[pl.ds(g * _NL, _NL)] = plsc.load_gather(py_v, [iv])
        cz_v[pl.ds(g * _NL, _NL)] = plsc.load_gather(pz_v, [iv])
        return 0

    lax.fori_loop(0, ngrp, cgrp, 0)
    pltpu.sync_copy(cx_v, cx_h.at[pl.ds(base_c, chunk)])
    pltpu.sync_copy(cy_v, cy_h.at[pl.ds(base_c, chunk)])
    pltpu.sync_copy(cz_v, cz_h.at[pl.ds(base_c, chunk)])

    def per_group(g, _):
      gbase = g * _NL

      def per_center(cl, _):
        c = gbase + cl
        cv = jnp.full((_NL,), c, dtype=jnp.int32)
        cxs = plsc.load_gather(cx_v, [cv])
        cys = plsc.load_gather(cy_v, [cv])
        czs = plsc.load_gather(cz_v, [cv])

        AUN = 4

        def phase_a(j4, _):
            for u in range(AUN):
                j = j4 * AUN + u
                pxj = px_v[pl.ds(j * _NL, _NL)]
                pyj = py_v[pl.ds(j * _NL, _NL)]
                pzj = pz_v[pl.ds(j * _NL, _NL)]
                dx = cxs - pxj
                dy = cys - pyj
                dz = czs - pzj
                d2 = (dx * dx + dy * dy) + dz * dz
                msk_v[pl.ds(j * _NL, _NL)] = (d2 <= r2).astype(jnp.int32)
            return 0

        lax.fori_loop(0, n // (_NL * AUN), phase_a, 0)

        BUN = 2
        cbase = cv * k

        def phase_b(j2, off):
            for u in range(BUN):
                j = j2 * BUN + u
                m16 = msk_v[pl.ds(j * _NL, _NL)]
                mb = m16 > 0
                cum = plsc.cumsum(m16)
                idx = off + (cum - m16)
                jv = j * _NL + iota
                plsc.store_scatter(nb_v, [cbase + idx], jv,
                                   mask=mb & (idx < k))
                off = off + plsc.all_reduce_population_count(mb)
            return off

        off = lax.fori_loop(0, n // (_NL * BUN), phase_b,
                            jnp.zeros((_NL,), jnp.int32))
        cnts = jnp.minimum(off, k)

        if True:
            for t in range(k // _NL):
                slot = t * _NL + iota
                nbv = nb_v[pl.ds(c * k + t * _NL, _NL)]
                nbv = jnp.minimum(jnp.maximum(nbv, 0), n - 1)
                gx = plsc.load_gather(px_v, [nbv])
                gy = plsc.load_gather(py_v, [nbv])
                gz = plsc.load_gather(pz_v, [nbv])
                if slot_major:
                    row16 = slot * dpad
                    clv = cv
                    plsc.store_scatter(stage_v, [row16 + cf, clv], gx - cxs)
                    plsc.store_scatter(stage_v, [row16 + (cf + 1), clv],
                                       gy - cys)
                    plsc.store_scatter(stage_v, [row16 + (cf + 2), clv],
                                       gz - czs)
                    for col in range(cf):
                        vals = plsc.load_gather(feat_v, [nbv * cf + col])
                        plsc.store_scatter(stage_v, [row16 + col, clv], vals)
                    for col in range(cf + 3, dpad):
                        plsc.store_scatter(stage_v, [row16 + col, clv],
                                           zeros16)
                    plsc.store_scatter(stvld_v, [slot, clv],
                                       (slot < cnts).astype(jnp.int32))
                else:
                    rowflat = (cl * k + t * _NL + iota) * D
                    plsc.store_scatter(stage_v, [rowflat + cf], gx - cxs)
                    plsc.store_scatter(stage_v, [rowflat + (cf + 1)],
                                       gy - cys)
                    plsc.store_scatter(stage_v, [rowflat + (cf + 2)],
                                       gz - czs)
                    if feat_cm:
                        def col_loop(cc, _):
                            for u in range(8):
                                col = cc * 8 + u
                                colv = jnp.full((_NL,), col, dtype=jnp.int32)
                                vals = plsc.load_gather(feat_v, [colv, nbv])
                                plsc.store_scatter(stage_v, [rowflat + col],
                                                   vals)
                            return 0
                        lax.fori_loop(0, cf // 8, col_loop, 0)
                    else:
                        for col in range(cf):
                            vals = plsc.load_gather(feat_v, [nbv * cf + col])
                            plsc.store_scatter(stage_v, [rowflat + col], vals)
                    stvld_v[pl.ds(cl * k + t * _NL, _NL)] = (
                        (slot < cnts).astype(jnp.int32))
        return 0

      lax.fori_loop(0, _NL, per_center, 0)

      if not slot_major:
          row0 = (base_c + gbase) * k
          pltpu.sync_copy(stage_v, mlp_h.at[pl.ds(row0 * D, _NL * k * D)])
          pltpu.sync_copy(stvld_v, valid_h.at[pl.ds(row0, _NL * k)])
      return 0

    lax.fori_loop(0, ngrp, per_group, 0)

    if slot_major:
        pltpu.sync_copy(stage_v, mlp_h.at[:, pl.ds(base_c, chunk)])
        pltpu.sync_copy(stvld_v, valid_h.at[:, pl.ds(base_c, chunk)])


def _sc_prep(px, py, pz, feat, idx, k, r2, slot_major, dpad=None):
    b, n = px.shape
    s = idx.shape[1]
    if slot_major:
        cf = feat.shape[1] // n
    else:
        cf = feat.shape[0]
    C = b * s
    D = cf + 3
    if dpad is None:
        dpad = D
    chunk = C // _NW
    mesh = plsc.VectorSubcoreMesh(core_axis_name="c", subcore_axis_name="s",
                                  num_cores=_NC, num_subcores=_NS)
    body = functools.partial(_sc_prep_body, b=b, n=n, s=s, k=k, cf=cf, r2=r2,
                             slot_major=slot_major,
                             feat_cm=(not slot_major), dpad=dpad)
    if slot_major:
        mlp_ty = jax.ShapeDtypeStruct((k * dpad, C), jnp.float32)
        vld_ty = jax.ShapeDtypeStruct((k, C), jnp.int32)
        stage_ty = pltpu.VMEM((k * dpad, chunk), jnp.float32)
        stvld_ty = pltpu.VMEM((k, chunk), jnp.int32)
        feat_v_ty = pltpu.VMEM((n * cf,), jnp.float32)
    else:
        mlp_ty = jax.ShapeDtypeStruct((C * k * D,), jnp.float32)
        vld_ty = jax.ShapeDtypeStruct((C * k,), jnp.int32)
        stage_ty = pltpu.VMEM((_NL * k * D,), jnp.float32)
        stvld_ty = pltpu.VMEM((_NL * k,), jnp.int32)
        feat_v_ty = pltpu.VMEM((cf, n), jnp.float32)
    f = pl.kernel(
        body,
        out_type=(
            mlp_ty,
            vld_ty,
            jax.ShapeDtypeStruct((C,), jnp.float32),
            jax.ShapeDtypeStruct((C,), jnp.float32),
            jax.ShapeDtypeStruct((C,), jnp.float32),
        ),
        mesh=mesh,
        compiler_params=pltpu.CompilerParams(needs_layout_passes=False),
        scratch_types=[
            pltpu.VMEM((n,), jnp.float32),
            pltpu.VMEM((n,), jnp.float32),
            pltpu.VMEM((n,), jnp.float32),
            feat_v_ty,
            pltpu.VMEM((chunk,), jnp.int32),
            pltpu.VMEM((chunk,), jnp.float32),
            pltpu.VMEM((chunk,), jnp.float32),
            pltpu.VMEM((chunk,), jnp.float32),
            pltpu.VMEM((n,), jnp.int32),
            pltpu.VMEM((chunk * k,), jnp.int32),
            stage_ty,
            stvld_ty,
        ],
    )
    return f(px, py, pz, feat, idx.reshape(C))


def _mlp_body_t(x_ref, v_ref, w1_ref, b1_ref, w2_ref, b2_ref, w3_ref, b3_ref,
                o1_ref, o2_ref, *, k, dpad):
    w1 = w1_ref[...]
    w2 = w2_ref[...]
    w3 = w3_ref[...]
    b1 = b1_ref[...]
    b2 = b2_ref[...]
    b3 = b3_ref[...]
    acc = None
    for s in range(k):
        xs = x_ref[s * dpad:(s + 1) * dpad, :]
        h = jnp.maximum(jnp.dot(w1, xs, preferred_element_type=jnp.float32)
                        + b1, 0.0)
        h = jnp.maximum(jnp.dot(w2, h, preferred_element_type=jnp.float32)
                        + b2, 0.0)
        h = jnp.maximum(jnp.dot(w3, h, preferred_element_type=jnp.float32)
                        + b3, 0.0)
        vs = v_ref[s:s + 1, :] > 0
        h = jnp.where(vs, h, 0.0)
        acc = h if acc is None else jnp.maximum(acc, h)
    o1_ref[...] = acc
    o2_ref[...] = acc.T


def _mlp_pool_t(mlp_in, valid, params, k, dpad):
    rows, C = mlp_in.shape
    w1, b1, w2, b2, w3, b3 = params
    h1 = w1.shape[1]
    h2 = w2.shape[1]
    dout = w3.shape[1]
    w1t = jnp.zeros((h1, dpad), jnp.float32).at[:, :w1.shape[0]].set(w1.T)
    w2t = w2.T
    w3t = w3.T
    cbl = 512
    grid = C // cbl
    return pl.pallas_call(
        functools.partial(_mlp_body_t, k=k, dpad=dpad),
        grid=(grid,),
        in_specs=[
            pl.BlockSpec((k * dpad, cbl), lambda i: (0, i)),
            pl.BlockSpec((k, cbl), lambda i: (0, i)),
            pl.BlockSpec(w1t.shape, lambda i: (0, 0)),
            pl.BlockSpec((h1, 1), lambda i: (0, 0)),
            pl.BlockSpec(w2t.shape, lambda i: (0, 0)),
            pl.BlockSpec((h2, 1), lambda i: (0, 0)),
            pl.BlockSpec(w3t.shape, lambda i: (0, 0)),
            pl.BlockSpec((dout, 1), lambda i: (0, 0)),
        ],
        out_specs=(
            pl.BlockSpec((dout, cbl), lambda i: (0, i)),
            pl.BlockSpec((cbl, dout), lambda i: (i, 0)),
        ),
        out_shape=(
            jax.ShapeDtypeStruct((dout, C), jnp.float32),
            jax.ShapeDtypeStruct((C, dout), jnp.float32),
        ),
    )(mlp_in, valid, w1t, b1.reshape(-1, 1), w2t, b2.reshape(-1, 1),
      w3t, b3.reshape(-1, 1))


def _mlp_body(x_ref, v_ref, w1_ref, b1_ref, w2_ref, b2_ref, w3_ref, b3_ref,
              o_ref, *, cb, k):
    x = x_ref[...]
    h = jnp.maximum(jnp.dot(x, w1_ref[...],
                            preferred_element_type=jnp.float32)
                    + b1_ref[...], 0.0)
    h = jnp.maximum(jnp.dot(h, w2_ref[...],
                            preferred_element_type=jnp.float32)
                    + b2_ref[...], 0.0)
    h = jnp.maximum(jnp.dot(h, w3_ref[...],
                            preferred_element_type=jnp.float32)
                    + b3_ref[...], 0.0)
    v = v_ref[...] > 0
    h = jnp.where(v, h, 0.0)
    dout = h.shape[-1]
    o_ref[...] = jnp.max(h.reshape(cb, k, dout), axis=1)


def _mlp_pool(mlp_in, valid, params, k):
    rows, D = mlp_in.shape
    C = rows // k
    w1, b1, w2, b2, w3, b3 = params
    dout = w3.shape[1]
    cb = 128
    grid = C // cb
    return pl.pallas_call(
        functools.partial(_mlp_body, cb=cb, k=k),
        grid=(grid,),
        in_specs=[
            pl.BlockSpec((cb * k, D), lambda i: (i, 0)),
            pl.BlockSpec((cb * k, 1), lambda i: (i, 0)),
            pl.BlockSpec(w1.shape, lambda i: (0, 0)),
            pl.BlockSpec((1, b1.shape[0]), lambda i: (0, 0)),
            pl.BlockSpec(w2.shape, lambda i: (0, 0)),
            pl.BlockSpec((1, b2.shape[0]), lambda i: (0, 0)),
            pl.BlockSpec(w3.shape, lambda i: (0, 0)),
            pl.BlockSpec((1, b3.shape[0]), lambda i: (0, 0)),
        ],
        out_specs=pl.BlockSpec((cb, dout), lambda i: (i, 0)),
        out_shape=jax.ShapeDtypeStruct((C, dout), jnp.float32),
    )(mlp_in, valid.reshape(rows, 1), w1, b1.reshape(1, -1),
      w2, b2.reshape(1, -1), w3, b3.reshape(1, -1))


def kernel(x, mask, W11, b11, W12, b12, W13, b13, W21, b21, W22, b22, W23, b23):
    b, n, _ = x.shape
    feats0, pos0 = _split(x, mask)
    batch0 = jnp.repeat(jnp.arange(b), n)
    px = pos0[:, 0].reshape(b, n)
    py = pos0[:, 1].reshape(b, n)
    pz = pos0[:, 2].reshape(b, n)

    s1 = int(n * _RATIO1)
    dpad1 = 8
    idx1 = _fps_pallas(px, py, pz, s1)
    mlp1, valid1, cx1, cy1, cz1 = _sc_prep(
        px, py, pz, feats0.reshape(b, n * 3), idx1, _K1, _R1 * _R1,
        slot_major=True, dpad=dpad1)
    f1t, f1 = _mlp_pool_t(mlp1, valid1, (W11, b11, W12, b12, W13, b13),
                          _K1, dpad1)

    s2 = int(s1 * _RATIO2)
    px2 = cx1.reshape(b, s1)
    py2 = cy1.reshape(b, s1)
    pz2 = cz1.reshape(b, s1)
    idx2 = _fps_pallas(px2, py2, pz2, s2)
    mlp2, valid2, cx2, cy2, cz2 = _sc_prep(
        px2, py2, pz2, f1t, idx2, _K2, _R2 * _R2, slot_major=False)
    D2 = f1t.shape[0] + 3
    f2 = _mlp_pool(mlp2.reshape(b * s2 * _K2, D2), valid2,
                   (W21, b21, W22, b22, W23, b23), _K2)

    c1 = jnp.stack([cx1, cy1, cz1], axis=-1)
    c2 = jnp.stack([cx2, cy2, cz2], axis=-1)
    batch1 = jnp.repeat(jnp.arange(b), s1)
    batch2 = jnp.repeat(jnp.arange(b), s2)
    return (feats0, pos0, batch0, f1, c1, batch1, f2, c2, batch2)


# speedup vs baseline: 1.1422x; 1.1422x over previous
import functools

import jax
import jax.numpy as jnp
from jax import lax
from jax.experimental import pallas as pl
from jax.experimental.pallas import tpu as pltpu
from jax.experimental.pallas import tpu_sc as plsc

_R1, _RATIO1, _K1 = 0.5, 0.5, 32
_R2, _RATIO2, _K2 = 1.0, 0.25, 32

_NC, _NS, _NL = 2, 16, 16
_NW = _NC * _NS


def _split_body(x_ref, m_ref, feat_ref, pos_ref):
    x = x_ref[...]
    sel = m_ref[...] > 0
    feat_ref[...] = jnp.where(sel, x[:, 3:], 0.0)
    pos_ref[...] = jnp.where(sel, x[:, :3], 0.0)


def _split(x, mask):
    b, n, f = x.shape
    x2 = x.reshape(b * n, f)
    m2 = mask.reshape(b * n, 1).astype(jnp.int32)
    feat, pos = pl.pallas_call(
        _split_body,
        out_shape=(
            jax.ShapeDtypeStruct((b * n, f - 3), x.dtype),
            jax.ShapeDtypeStruct((b * n, 3), x.dtype),
        ),
    )(x2, m2)
    return feat, pos


def _fps_body(px_ref, py_ref, pz_ref, out_ref, *, s):
    px = px_ref[...]
    py = py_ref[...]
    pz = pz_ref[...]
    b, n = px.shape
    lane = lax.broadcasted_iota(jnp.int32, (b, n), 1)
    lane_s = lax.broadcasted_iota(jnp.int32, (b, s), 1)

    def step(t, carry):
        d, lx, ly, lz, acc = carry
        dx = px - lx
        dy = py - ly
        dz = pz - lz
        sq = (dx * dx + dy * dy) + dz * dz
        nd = jnp.minimum(d, sq)
        m = jnp.max(nd, axis=1, keepdims=True)
        cand = jnp.where(nd == m, lane, n)
        nxt = jnp.min(cand, axis=1, keepdims=True)
        acc = jnp.where(lane_s == t, nxt, acc)
        oh = lane == nxt
        nlx = jnp.sum(jnp.where(oh, px, 0.0), axis=1, keepdims=True)
        nly = jnp.sum(jnp.where(oh, py, 0.0), axis=1, keepdims=True)
        nlz = jnp.sum(jnp.where(oh, pz, 0.0), axis=1, keepdims=True)
        return nd, nlx, nly, nlz, acc

    def step2(t2, carry):
        carry = step(t2 * 2 + 1, carry)
        return step(t2 * 2 + 2, carry)

    init = (
        jnp.full((b, n), jnp.inf, dtype=jnp.float32),
        px[:, 0:1], py[:, 0:1], pz[:, 0:1],
        jnp.zeros((b, s), dtype=jnp.int32),
    )
    carry = lax.fori_loop(0, (s - 2) // 2, step2, init)
    carry = step(s - 1, carry)
    out_ref[...] = carry[4]


def _fps_pallas(px, py, pz, s):
    b, n = px.shape
    return pl.pallas_call(
        functools.partial(_fps_body, s=s),
        out_shape=jax.ShapeDtypeStruct((b, s), jnp.int32),
    )(px, py, pz)


def _sc_prep_body(px_h, py_h, pz_h, feat_h, idx_h,
                  mlp_h, valid_h, cx_h, cy_h, cz_h,
                  px_v, py_v, pz_v, feat_v, idx_v,
                  cx_v, cy_v, cz_v, msk_v, chunkcnt_v, visit_v, nb_v,
                  stage_v, stvld_v,
                  *, b, n, s, k, cf, r2, slot_major, feat_cm, dpad):
    D = cf + 3
    C = b * s
    chunk = C // _NW
    wpb = _NW // b
    ngrp = chunk // _NL

    cidx = lax.axis_index("c")
    sidx = lax.axis_index("s")
    wid = sidx * _NC + cidx
    bid = wid // wpb
    base_c = wid * chunk

    pltpu.sync_copy(px_h.at[bid], px_v)
    pltpu.sync_copy(py_h.at[bid], py_v)
    pltpu.sync_copy(pz_h.at[bid], pz_v)
    if feat_cm:
        pltpu.sync_copy(feat_h.at[:, pl.ds(bid * n, n)], feat_v)
    else:
        pltpu.sync_copy(feat_h.at[bid], feat_v)
    pltpu.sync_copy(idx_h.at[pl.ds(base_c, chunk)], idx_v)

    iota = lax.iota(jnp.int32, _NL)
    zeros16 = jnp.zeros((_NL,), jnp.float32)

    def cgrp(g, _):
        iv = idx_v[pl.ds(g * _NL, _NL)]
        cx_v[pl.ds(g * _NL, _NL)] = plsc.load_gather(px_v, [iv])
        cy_v[pl.ds(g * _NL, _NL)] = plsc.load_gather(py_v, [iv])
        cz_v[pl.ds(g * _NL, _NL)] = plsc.load_gather(pz_v, [iv])
        return 0

    lax.fori_loop(0, ngrp, cgrp, 0)
    pltpu.sync_copy(cx_v, cx_h.at[pl.ds(base_c, chunk)])
    pltpu.sync_copy(cy_v, cy_h.at[pl.ds(base_c, chunk)])
    pltpu.sync_copy(cz_v, cz_h.at[pl.ds(base_c, chunk)])

    def per_group(g, _):
      gbase = g * _NL

      def per_center(cl, _):
        c = gbase + cl
        cv = jnp.full((_NL,), c, dtype=jnp.int32)
        cxs = plsc.load_gather(cx_v, [cv])
        cys = plsc.load_gather(cy_v, [cv])
        czs = plsc.load_gather(cz_v, [cv])

        NCH = n // _NL

        def phase_a(m, _):
            hit = jnp.zeros((_NL,), jnp.int32)
            for u in range(_NL):
                j = m * _NL + u
                pxj = px_v[pl.ds(j * _NL, _NL)]
                pyj = py_v[pl.ds(j * _NL, _NL)]
                pzj = pz_v[pl.ds(j * _NL, _NL)]
                dx = cxs - pxj
                dy = cys - pyj
                dz = czs - pzj
                d2 = (dx * dx + dy * dy) + dz * dz
                mb = d2 <= r2
                msk_v[pl.ds(j * _NL, _NL)] = mb.astype(jnp.int32)
                hit = jnp.where(iota == u,
                                plsc.all_reduce_population_count(mb), hit)
            chunkcnt_v[pl.ds(m * _NL, _NL)] = hit
            return 0

        lax.fori_loop(0, NCH // _NL, phase_a, 0)

        def build_visit(v, voff):
            c16 = chunkcnt_v[pl.ds(v * _NL, _NL)]
            mbi = (c16 > 0).astype(jnp.int32)
            mb = c16 > 0
            cum = plsc.cumsum(mbi)
            idxv = voff + (cum - mbi)
            plsc.store_scatter(visit_v, [idxv], v * _NL + iota, mask=mb)
            return voff + plsc.all_reduce_population_count(mb)

        voff = lax.fori_loop(0, NCH // _NL, build_visit,
                             jnp.zeros((_NL,), jnp.int32))
        nvisit = jnp.max(voff)

        cbase = cv * k

        def phase_b(vi, off):
            vsp = plsc.load_gather(visit_v, [jnp.full((_NL,), vi, jnp.int32)])
            jv = vsp * _NL + iota
            m16 = plsc.load_gather(msk_v, [jv])
            mb = m16 > 0
            cum = plsc.cumsum(m16)
            idx = off + (cum - m16)
            plsc.store_scatter(nb_v, [cbase + idx], jv,
                               mask=mb & (idx < k))
            return off + plsc.all_reduce_population_count(mb)

        off = lax.fori_loop(0, nvisit, phase_b, jnp.zeros((_NL,), jnp.int32))
        cnts = jnp.minimum(off, k)

        if True:
            for t in range(k // _NL):
                slot = t * _NL + iota
                nbv = nb_v[pl.ds(c * k + t * _NL, _NL)]
                nbv = jnp.minimum(jnp.maximum(nbv, 0), n - 1)
                gx = plsc.load_gather(px_v, [nbv])
                gy = plsc.load_gather(py_v, [nbv])
                gz = plsc.load_gather(pz_v, [nbv])
                if slot_major:
                    row16 = slot * dpad
                    clv = cv
                    plsc.store_scatter(stage_v, [row16 + cf, clv], gx - cxs)
                    plsc.store_scatter(stage_v, [row16 + (cf + 1), clv],
                                       gy - cys)
                    plsc.store_scatter(stage_v, [row16 + (cf + 2), clv],
                                       gz - czs)
                    for col in range(cf):
                        vals = plsc.load_gather(feat_v, [nbv * cf + col])
                        plsc.store_scatter(stage_v, [row16 + col, clv], vals)
                    for col in range(cf + 3, dpad):
                        plsc.store_scatter(stage_v, [row16 + col, clv],
                                           zeros16)
                    plsc.store_scatter(stvld_v, [slot, clv],
                                       (slot < cnts).astype(jnp.int32))
                else:
                    rowflat = (cl * k + t * _NL + iota) * D
                    plsc.store_scatter(stage_v, [rowflat + cf], gx - cxs)
                    plsc.store_scatter(stage_v, [rowflat + (cf + 1)],
                                       gy - cys)
                    plsc.store_scatter(stage_v, [rowflat + (cf + 2)],
                                       gz - czs)
                    if feat_cm:
                        def col_loop(cc, _):
                            for u in range(8):
                                col = cc * 8 + u
                                colv = jnp.full((_NL,), col, dtype=jnp.int32)
                                vals = plsc.load_gather(feat_v, [colv, nbv])
                                plsc.store_scatter(stage_v, [rowflat + col],
                                                   vals)
                            return 0
                        lax.fori_loop(0, cf // 8, col_loop, 0)
                    else:
                        for col in range(cf):
                            vals = plsc.load_gather(feat_v, [nbv * cf + col])
                            plsc.store_scatter(stage_v, [rowflat + col], vals)
                    stvld_v[pl.ds(cl * k + t * _NL, _NL)] = (
                        (slot < cnts).astype(jnp.int32))
        return 0

      lax.fori_loop(0, _NL, per_center, 0)

      if not slot_major:
          row0 = (base_c + gbase) * k
          pltpu.sync_copy(stage_v, mlp_h.at[pl.ds(row0 * D, _NL * k * D)])
          pltpu.sync_copy(stvld_v, valid_h.at[pl.ds(row0, _NL * k)])
      return 0

    lax.fori_loop(0, ngrp, per_group, 0)

    if slot_major:
        pltpu.sync_copy(stage_v, mlp_h.at[:, pl.ds(base_c, chunk)])
        pltpu.sync_copy(stvld_v, valid_h.at[:, pl.ds(base_c, chunk)])


def _sc_prep(px, py, pz, feat, idx, k, r2, slot_major, dpad=None):
    b, n = px.shape
    s = idx.shape[1]
    if slot_major:
        cf = feat.shape[1] // n
    else:
        cf = feat.shape[0]
    C = b * s
    D = cf + 3
    if dpad is None:
        dpad = D
    chunk = C // _NW
    mesh = plsc.VectorSubcoreMesh(core_axis_name="c", subcore_axis_name="s",
                                  num_cores=_NC, num_subcores=_NS)
    body = functools.partial(_sc_prep_body, b=b, n=n, s=s, k=k, cf=cf, r2=r2,
                             slot_major=slot_major,
                             feat_cm=(not slot_major), dpad=dpad)
    if slot_major:
        mlp_ty = jax.ShapeDtypeStruct((k * dpad, C), jnp.float32)
        vld_ty = jax.ShapeDtypeStruct((k, C), jnp.int32)
        stage_ty = pltpu.VMEM((k * dpad, chunk), jnp.float32)
        stvld_ty = pltpu.VMEM((k, chunk), jnp.int32)
        feat_v_ty = pltpu.VMEM((n * cf,), jnp.float32)
    else:
        mlp_ty = jax.ShapeDtypeStruct((C * k * D,), jnp.float32)
        vld_ty = jax.ShapeDtypeStruct((C * k,), jnp.int32)
        stage_ty = pltpu.VMEM((_NL * k * D,), jnp.float32)
        stvld_ty = pltpu.VMEM((_NL * k,), jnp.int32)
        feat_v_ty = pltpu.VMEM((cf, n), jnp.float32)
    f = pl.kernel(
        body,
        out_type=(
            mlp_ty,
            vld_ty,
            jax.ShapeDtypeStruct((C,), jnp.float32),
            jax.ShapeDtypeStruct((C,), jnp.float32),
            jax.ShapeDtypeStruct((C,), jnp.float32),
        ),
        mesh=mesh,
        compiler_params=pltpu.CompilerParams(needs_layout_passes=False),
        scratch_types=[
            pltpu.VMEM((n,), jnp.float32),
            pltpu.VMEM((n,), jnp.float32),
            pltpu.VMEM((n,), jnp.float32),
            feat_v_ty,
            pltpu.VMEM((chunk,), jnp.int32),
            pltpu.VMEM((chunk,), jnp.float32),
            pltpu.VMEM((chunk,), jnp.float32),
            pltpu.VMEM((chunk,), jnp.float32),
            pltpu.VMEM((n,), jnp.int32),
            pltpu.VMEM((n // _NL,), jnp.int32),
            pltpu.VMEM((n // _NL,), jnp.int32),
            pltpu.VMEM((chunk * k,), jnp.int32),
            stage_ty,
            stvld_ty,
        ],
    )
    return f(px, py, pz, feat, idx.reshape(C))


def _mlp_body_t(x_ref, v_ref, w1_ref, b1_ref, w2_ref, b2_ref, w3_ref, b3_ref,
                o1_ref, o2_ref, *, k, dpad):
    w1 = w1_ref[...]
    w2 = w2_ref[...]
    w3 = w3_ref[...]
    b1 = b1_ref[...]
    b2 = b2_ref[...]
    b3 = b3_ref[...]
    acc = None
    for s in range(k):
        xs = x_ref[s * dpad:(s + 1) * dpad, :]
        h = jnp.maximum(jnp.dot(w1, xs, preferred_element_type=jnp.float32)
                        + b1, 0.0)
        h = jnp.maximum(jnp.dot(w2, h, preferred_element_type=jnp.float32)
                        + b2, 0.0)
        h = jnp.maximum(jnp.dot(w3, h, preferred_element_type=jnp.float32)
                        + b3, 0.0)
        vs = v_ref[s:s + 1, :] > 0
        h = jnp.where(vs, h, 0.0)
        acc = h if acc is None else jnp.maximum(acc, h)
    o1_ref[...] = acc
    o2_ref[...] = acc.T


def _mlp_pool_t(mlp_in, valid, params, k, dpad):
    rows, C = mlp_in.shape
    w1, b1, w2, b2, w3, b3 = params
    h1 = w1.shape[1]
    h2 = w2.shape[1]
    dout = w3.shape[1]
    w1t = jnp.zeros((h1, dpad), jnp.float32).at[:, :w1.shape[0]].set(w1.T)
    w2t = w2.T
    w3t = w3.T
    cbl = 512
    grid = C // cbl
    return pl.pallas_call(
        functools.partial(_mlp_body_t, k=k, dpad=dpad),
        grid=(grid,),
        in_specs=[
            pl.BlockSpec((k * dpad, cbl), lambda i: (0, i)),
            pl.BlockSpec((k, cbl), lambda i: (0, i)),
            pl.BlockSpec(w1t.shape, lambda i: (0, 0)),
            pl.BlockSpec((h1, 1), lambda i: (0, 0)),
            pl.BlockSpec(w2t.shape, lambda i: (0, 0)),
            pl.BlockSpec((h2, 1), lambda i: (0, 0)),
            pl.BlockSpec(w3t.shape, lambda i: (0, 0)),
            pl.BlockSpec((dout, 1), lambda i: (0, 0)),
        ],
        out_specs=(
            pl.BlockSpec((dout, cbl), lambda i: (0, i)),
            pl.BlockSpec((cbl, dout), lambda i: (i, 0)),
        ),
        out_shape=(
            jax.ShapeDtypeStruct((dout, C), jnp.float32),
            jax.ShapeDtypeStruct((C, dout), jnp.float32),
        ),
    )(mlp_in, valid, w1t, b1.reshape(-1, 1), w2t, b2.reshape(-1, 1),
      w3t, b3.reshape(-1, 1))


def _mlp_body(x_ref, v_ref, w1_ref, b1_ref, w2_ref, b2_ref, w3_ref, b3_ref,
              o_ref, *, cb, k):
    x = x_ref[...]
    h = jnp.maximum(jnp.dot(x, w1_ref[...],
                            preferred_element_type=jnp.float32)
                    + b1_ref[...], 0.0)
    h = jnp.maximum(jnp.dot(h, w2_ref[...],
                            preferred_element_type=jnp.float32)
                    + b2_ref[...], 0.0)
    h = jnp.maximum(jnp.dot(h, w3_ref[...],
                            preferred_element_type=jnp.float32)
                    + b3_ref[...], 0.0)
    v = v_ref[...] > 0
    h = jnp.where(v, h, 0.0)
    dout = h.shape[-1]
    o_ref[...] = jnp.max(h.reshape(cb, k, dout), axis=1)


def _mlp_pool(mlp_in, valid, params, k):
    rows, D = mlp_in.shape
    C = rows // k
    w1, b1, w2, b2, w3, b3 = params
    dout = w3.shape[1]
    cb = 128
    grid = C // cb
    return pl.pallas_call(
        functools.partial(_mlp_body, cb=cb, k=k),
        grid=(grid,),
        in_specs=[
            pl.BlockSpec((cb * k, D), lambda i: (i, 0)),
            pl.BlockSpec((cb * k, 1), lambda i: (i, 0)),
            pl.BlockSpec(w1.shape, lambda i: (0, 0)),
            pl.BlockSpec((1, b1.shape[0]), lambda i: (0, 0)),
            pl.BlockSpec(w2.shape, lambda i: (0, 0)),
            pl.BlockSpec((1, b2.shape[0]), lambda i: (0, 0)),
            pl.BlockSpec(w3.shape, lambda i: (0, 0)),
            pl.BlockSpec((1, b3.shape[0]), lambda i: (0, 0)),
        ],
        out_specs=pl.BlockSpec((cb, dout), lambda i: (i, 0)),
        out_shape=jax.ShapeDtypeStruct((C, dout), jnp.float32),
    )(mlp_in, valid.reshape(rows, 1), w1, b1.reshape(1, -1),
      w2, b2.reshape(1, -1), w3, b3.reshape(1, -1))


def kernel(x, mask, W11, b11, W12, b12, W13, b13, W21, b21, W22, b22, W23, b23):
    b, n, _ = x.shape
    feats0, pos0 = _split(x, mask)
    batch0 = jnp.repeat(jnp.arange(b), n)
    px = pos0[:, 0].reshape(b, n)
    py = pos0[:, 1].reshape(b, n)
    pz = pos0[:, 2].reshape(b, n)

    s1 = int(n * _RATIO1)
    dpad1 = 8
    idx1 = _fps_pallas(px, py, pz, s1)
    mlp1, valid1, cx1, cy1, cz1 = _sc_prep(
        px, py, pz, feats0.reshape(b, n * 3), idx1, _K1, _R1 * _R1,
        slot_major=True, dpad=dpad1)
    f1t, f1 = _mlp_pool_t(mlp1, valid1, (W11, b11, W12, b12, W13, b13),
                          _K1, dpad1)

    s2 = int(s1 * _RATIO2)
    px2 = cx1.reshape(b, s1)
    py2 = cy1.reshape(b, s1)
    pz2 = cz1.reshape(b, s1)
    idx2 = _fps_pallas(px2, py2, pz2, s2)
    mlp2, valid2, cx2, cy2, cz2 = _sc_prep(
        px2, py2, pz2, f1t, idx2, _K2, _R2 * _R2, slot_major=False)
    D2 = f1t.shape[0] + 3
    f2 = _mlp_pool(mlp2.reshape(b * s2 * _K2, D2), valid2,
                   (W21, b21, W22, b22, W23, b23), _K2)

    c1 = jnp.stack([cx1, cy1, cz1], axis=-1)
    c2 = jnp.stack([cx2, cy2, cz2], axis=-1)
    batch1 = jnp.repeat(jnp.arange(b), s1)
    batch2 = jnp.repeat(jnp.arange(b), s2)
    return (feats0, pos0, batch0, f1, c1, batch1, f2, c2, batch2)
